# baseline (speedup 1.0000x reference)
import dataclasses
import functools

import jax
import jax.numpy as jnp
from jax import lax
from jax.experimental import pallas as pl
from jax.experimental.pallas import tpu as pltpu
from jax.experimental.pallas import tpu_sc as plsc


def _edge_w_body(ef_ref, w1_ref, b1_ref, w2t_ref, b2_ref, o_ref):
    ef = ef_ref[...].astype(jnp.bfloat16)
    w1 = w1_ref[...].astype(jnp.bfloat16)
    b1 = b1_ref[...].astype(jnp.bfloat16)
    h = (ef[:, 0:1] * w1[0:1, :]
         + ef[:, 1:2] * w1[1:2, :]
         + ef[:, 2:3] * w1[2:3, :]
         + b1)
    h = jnp.maximum(h, jnp.bfloat16(0))
    hw = h * w2t_ref[...].astype(jnp.bfloat16)
    s = jnp.sum(hw.astype(jnp.float32), axis=1, keepdims=True) + b2_ref[...]
    o_ref[...] = jax.nn.sigmoid(s)


def _edge_weights(ef, w1, b1r, w2t, b2c):
    E, _ = ef.shape
    H = w1.shape[1]
    BE = 3200
    grid = E // BE
    return pl.pallas_call(
        _edge_w_body,
        grid=(grid,),
        in_specs=[
            pl.BlockSpec((BE, 3), lambda i: (i, 0)),
            pl.BlockSpec((3, H), lambda i: (0, 0)),
            pl.BlockSpec((1, H), lambda i: (0, 0)),
            pl.BlockSpec((1, H), lambda i: (0, 0)),
            pl.BlockSpec((1, 1), lambda i: (0, 0)),
        ],
        out_specs=pl.BlockSpec((BE, 1), lambda i: (i, 0)),
        out_shape=jax.ShapeDtypeStruct((E, 1), jnp.float32),
    )(ef, w1, b1r, w2t, b2c)


def _sc_message_pass(src3, dst3, w3, x2):
    NS, n_chunks, CH = src3.shape
    NC, N, DH = x2.shape
    NP = 10240
    rows_per_tile = NP // NS
    ZR = 160
    NJ = DH // 16

    mesh = plsc.VectorSubcoreMesh(core_axis_name="c", subcore_axis_name="s")
    cp = pltpu.CompilerParams()
    for fld, val in (("needs_layout_passes", False),
                     ("use_tc_tiling_on_sc", False)):
        if fld in pltpu.CompilerParams.__dataclass_fields__:
            cp = dataclasses.replace(cp, **{fld: val})

    KG = 5
    G = n_chunks // KG
    assert n_chunks % KG == 0

    @functools.partial(
        pl.kernel,
        out_type=jax.ShapeDtypeStruct((NC, NP, DH), jnp.float32),
        mesh=mesh,
        compiler_params=cp,
        scratch_types=[
            pltpu.VMEM((2, KG, CH), jnp.int32),
            pltpu.VMEM((n_chunks, CH), jnp.int32),
            pltpu.VMEM((2, KG, CH), jnp.float32),
            pltpu.VMEM((2, KG, CH, DH), jnp.float32),
            pltpu.VMEM((ZR, DH), jnp.float32),
            pltpu.VMEM_SHARED((NP, DH), jnp.float32),
            pltpu.SemaphoreType.DMA,
            pltpu.SemaphoreType.DMA,
            pltpu.SemaphoreType.DMA,
            pltpu.SemaphoreType.DMA,
        ],
    )
    def sck(src_hbm, dst_hbm, w_hbm, x_hbm, out_hbm,
            sidx, didx, wv, rows, zbuf, aggsh, gsem0, gsem1, ssem0, ssem1):
        cid = lax.axis_index("c")
        sid = lax.axis_index("s")
        zv = jnp.zeros((16,), jnp.float32)
        gsems = (gsem0, gsem1)
        ssems = (ssem0, ssem1)

        @pl.loop(0, ZR)
        def _(i):
            for j in range(NJ):
                zbuf[i, pl.ds(j * 16, 16)] = zv

        @pl.loop(0, rows_per_tile // ZR)
        def _(k):
            pltpu.sync_copy(zbuf, aggsh.at[pl.ds(sid * rows_per_tile + k * ZR, ZR)])

        plsc.subcore_barrier()

        pltpu.sync_copy(dst_hbm.at[sid], didx)

        def fire(g, slot, sem):
            pltpu.async_copy(w_hbm.at[sid, pl.ds(g * KG, KG)], wv.at[slot], sem)
            pltpu.async_copy(src_hbm.at[sid, pl.ds(g * KG, KG)],
                             sidx.at[slot], sem)
            for c in range(KG):
                pltpu.async_copy(x_hbm.at[cid].at[didx.at[g * KG + c]],
                                 rows.at[slot, c], sem)

        def drain(slot, sem):
            pltpu.make_async_copy(w_hbm.at[0, pl.ds(0, KG)],
                                  wv.at[slot], sem).wait()
            pltpu.make_async_copy(src_hbm.at[0, pl.ds(0, KG)],
                                  sidx.at[slot], sem).wait()
            for c in range(KG):
                pltpu.make_async_copy(x_hbm.at[0, pl.ds(0, CH)],
                                      rows.at[slot, c], sem).wait()

        def drain_scatter(slot):
            for c in range(KG):
                pltpu.make_async_copy(rows.at[slot, c],
                                      aggsh.at[sidx.at[slot, c]],
                                      ssems[slot]).wait()

        def scale_and_scatter(slot):
            for c in range(KG):
                @pl.loop(0, CH, step=2)
                def _(i):
                    for u in range(2):
                        wspl = plsc.load_gather(
                            wv, [jnp.full((16,), slot, jnp.int32),
                                 jnp.full((16,), c, jnp.int32),
                                 jnp.full((16,), i + u, jnp.int32)])
                        for j in range(NJ):
                            sl = pl.ds(j * 16, 16)
                            rows[slot, c, i + u, sl] = \
                                rows[slot, c, i + u, sl] * wspl

                pltpu.async_copy(rows.at[slot, c],
                                 aggsh.at[sidx.at[slot, c]],
                                 ssems[slot], add=True)

        fire(0, 0, gsem0)

        @pl.loop(0, G // 2)
        def _(gg):
            for slot in (0, 1):
                g = gg * 2 + slot
                drain(slot, gsems[slot])

                @pl.when(g >= 1)
                def _():
                    drain_scatter(1 - slot)

                @pl.when(g + 1 < G)
                def _():
                    fire(g + 1, 1 - slot, gsems[1 - slot])

                scale_and_scatter(slot)

        if G % 2 == 1:
            drain(0, gsem0)
            if G > 1:
                drain_scatter(1)
            scale_and_scatter(0)

        drain_scatter((G - 1) % 2)
        plsc.subcore_barrier()

        pltpu.sync_copy(
            aggsh.at[pl.ds(sid * rows_per_tile, rows_per_tile)],
            out_hbm.at[cid, pl.ds(sid * rows_per_tile, rows_per_tile)])

    return sck(src3, dst3, w3, x2)


def _node_body(x_ref, apa_ref, apb_ref, deg_ref, w1x_ref, w1al_ref, w1ar_ref,
               w1d_ref, bm1_ref, w2_ref, bm2_ref, g_ref, b_ref, o_ref):
    xb = x_ref[...]
    deg = deg_ref[...]
    inv = 1.0 / jnp.maximum(deg, 1.0)
    aggl = (apa_ref[0] + apb_ref[0]) * inv
    aggr = (apa_ref[1] + apb_ref[1]) * inv
    h = (jnp.dot(xb, w1x_ref[...], preferred_element_type=jnp.float32)
         + jnp.dot(aggl, w1al_ref[...], preferred_element_type=jnp.float32)
         + jnp.dot(aggr, w1ar_ref[...], preferred_element_type=jnp.float32)
         + deg * w1d_ref[...] + bm1_ref[...])
    h = jnp.maximum(h, 0.0)
    out = jnp.dot(h, w2_ref[...], preferred_element_type=jnp.float32) + bm2_ref[...]
    y = xb + out
    mean = jnp.mean(y, axis=1, keepdims=True)
    yc = y - mean
    var = jnp.mean(yc * yc, axis=1, keepdims=True)
    o_ref[...] = yc * lax.rsqrt(var + 1e-5) * g_ref[...] + b_ref[...]


def _node_mlp(x, aggpa, aggpb, deg2, w1x, w1al, w1ar, w1d, bm1, w2, bm2,
              gamma, beta):
    N, D = x.shape
    DH = aggpa.shape[2]
    H = w1x.shape[1]
    BN = 2000
    grid = N // BN
    full = lambda shape: pl.BlockSpec(shape, lambda i: (0, 0))
    return pl.pallas_call(
        _node_body,
        grid=(grid,),
        in_specs=[
            pl.BlockSpec((BN, D), lambda i: (i, 0)),
            pl.BlockSpec((2, BN, DH), lambda i: (0, i, 0)),
            pl.BlockSpec((2, BN, DH), lambda i: (0, i, 0)),
            pl.BlockSpec((BN, 1), lambda i: (i, 0)),
            full((D, H)), full((DH, H)), full((DH, H)), full((1, H)),
            full((1, H)), full((H, D)), full((1, D)), full((1, D)), full((1, D)),
        ],
        out_specs=pl.BlockSpec((BN, D), lambda i: (i, 0)),
        out_shape=jax.ShapeDtypeStruct((N, D), jnp.float32),
    )(x, aggpa, aggpb, deg2, w1x, w1al, w1ar, w1d, bm1, w2, bm2, gamma, beta)


def kernel(x, edge_index, edge_feat, degrees,
           W1, b1, W2, b2, Wm1, bm1, Wm2, bm2, gamma, beta):
    N, D = x.shape
    E = edge_index.shape[1]
    H = W1.shape[1]
    NS = 16
    E2 = E // 2
    per_t = E2 // NS
    CH = 80
    n_chunks = per_t // CH
    DH = D // 2

    x2 = x.reshape(N, 2, DH).transpose(1, 0, 2)
    b1r = b1.reshape(1, H)
    w2t = W2.reshape(1, H)
    b2c = b2.reshape(1, 1)

    aggps = []
    for ph in range(2):
        efp = lax.slice_in_dim(edge_feat, ph * E2, (ph + 1) * E2, axis=0)
        w = _edge_weights(efp, W1, b1r, w2t, b2c)
        src3 = lax.slice_in_dim(edge_index[0], ph * E2,
                                (ph + 1) * E2).reshape(NS, n_chunks, CH)
        dst3 = lax.slice_in_dim(edge_index[1], ph * E2,
                                (ph + 1) * E2).reshape(NS, n_chunks, CH)
        w3 = w.reshape(NS, n_chunks, CH)
        aggps.append(_sc_message_pass(src3, dst3, w3, x2))

    y = _node_mlp(x, aggps[0], aggps[1], degrees.reshape(N, 1),
                  Wm1[:D], Wm1[D:D + DH], Wm1[D + DH:2 * D], Wm1[2 * D:],
                  bm1.reshape(1, H), Wm2, bm2.reshape(1, D),
                  gamma.reshape(1, D), beta.reshape(1, D))
    return y


# speedup vs baseline: 1.4854x; 1.4854x over previous
import dataclasses
import functools

import jax
import jax.numpy as jnp
from jax import lax
from jax.experimental import pallas as pl
from jax.experimental.pallas import tpu as pltpu
from jax.experimental.pallas import tpu_sc as plsc


def _edge_w_body(eft_ref, w1t_ref, b1_ref, w2_ref, b2_ref, o_ref):
    ef = eft_ref[...].astype(jnp.bfloat16)
    w1t = w1t_ref[...].astype(jnp.bfloat16)
    h = (w1t[:, 0:1] * ef[0:1, :]
         + w1t[:, 1:2] * ef[1:2, :]
         + w1t[:, 2:3] * ef[2:3, :]
         + b1_ref[...].astype(jnp.bfloat16))
    h = jnp.maximum(h, jnp.bfloat16(0))
    hw = h * w2_ref[...].astype(jnp.bfloat16)
    s = jnp.sum(hw.astype(jnp.float32), axis=0, keepdims=True) + b2_ref[...]
    o_ref[...] = jax.nn.sigmoid(s)


def _edge_weights(eft, w1t, b1c, w2, b2c):
    H, E = w1t.shape[0], eft.shape[1]
    BE = 3200
    grid = E // BE
    return pl.pallas_call(
        _edge_w_body,
        grid=(grid,),
        in_specs=[
            pl.BlockSpec((3, BE), lambda i: (0, i)),
            pl.BlockSpec((H, 3), lambda i: (0, 0)),
            pl.BlockSpec((H, 1), lambda i: (0, 0)),
            pl.BlockSpec((H, 1), lambda i: (0, 0)),
            pl.BlockSpec((1, 1), lambda i: (0, 0)),
        ],
        out_specs=pl.BlockSpec((1, BE), lambda i: (0, i)),
        out_shape=jax.ShapeDtypeStruct((1, E), jnp.float32),
    )(eft, w1t, b1c, w2, b2c)


def _sc_message_pass(src3, dst3, w3, x2):
    NS, n_chunks, CH = src3.shape
    NC, N, DH = x2.shape
    NP = 10240
    rows_per_tile = NP // NS
    ZR = 160
    NJ = DH // 16

    mesh = plsc.VectorSubcoreMesh(core_axis_name="c", subcore_axis_name="s")
    cp = pltpu.CompilerParams()
    for fld, val in (("needs_layout_passes", False),
                     ("use_tc_tiling_on_sc", False)):
        if fld in pltpu.CompilerParams.__dataclass_fields__:
            cp = dataclasses.replace(cp, **{fld: val})

    KG = 5
    G = n_chunks // KG
    assert n_chunks % KG == 0

    @functools.partial(
        pl.kernel,
        out_type=jax.ShapeDtypeStruct((NC, NP, DH), jnp.float32),
        mesh=mesh,
        compiler_params=cp,
        scratch_types=[
            pltpu.VMEM((2, KG, CH), jnp.int32),
            pltpu.VMEM((n_chunks, CH), jnp.int32),
            pltpu.VMEM((2, KG, CH), jnp.float32),
            pltpu.VMEM((2, KG, CH, DH), jnp.float32),
            pltpu.VMEM((ZR, DH), jnp.float32),
            pltpu.VMEM_SHARED((NP, DH), jnp.float32),
            pltpu.SemaphoreType.DMA,
            pltpu.SemaphoreType.DMA,
            pltpu.SemaphoreType.DMA,
            pltpu.SemaphoreType.DMA,
        ],
    )
    def sck(src_hbm, dst_hbm, w_hbm, x_hbm, out_hbm,
            sidx, didx, wv, rows, zbuf, aggsh, gsem0, gsem1, ssem0, ssem1):
        cid = lax.axis_index("c")
        sid = lax.axis_index("s")
        zv = jnp.zeros((16,), jnp.float32)
        gsems = (gsem0, gsem1)
        ssems = (ssem0, ssem1)

        @pl.loop(0, ZR)
        def _(i):
            for j in range(NJ):
                zbuf[i, pl.ds(j * 16, 16)] = zv

        @pl.loop(0, rows_per_tile // ZR)
        def _(k):
            pltpu.sync_copy(zbuf, aggsh.at[pl.ds(sid * rows_per_tile + k * ZR, ZR)])

        plsc.subcore_barrier()

        pltpu.sync_copy(dst_hbm.at[sid], didx)

        def fire(g, slot, sem):
            pltpu.async_copy(w_hbm.at[sid, pl.ds(g * KG, KG)], wv.at[slot], sem)
            pltpu.async_copy(src_hbm.at[sid, pl.ds(g * KG, KG)],
                             sidx.at[slot], sem)
            for c in range(KG):
                pltpu.async_copy(x_hbm.at[cid].at[didx.at[g * KG + c]],
                                 rows.at[slot, c], sem)

        def drain(slot, sem):
            pltpu.make_async_copy(w_hbm.at[0, pl.ds(0, KG)],
                                  wv.at[slot], sem).wait()
            pltpu.make_async_copy(src_hbm.at[0, pl.ds(0, KG)],
                                  sidx.at[slot], sem).wait()
            for c in range(KG):
                pltpu.make_async_copy(x_hbm.at[0, pl.ds(0, CH)],
                                      rows.at[slot, c], sem).wait()

        def drain_scatter(slot):
            for c in range(KG):
                pltpu.make_async_copy(rows.at[slot, c],
                                      aggsh.at[sidx.at[slot, c]],
                                      ssems[slot]).wait()

        def scale_and_scatter(slot):
            for c in range(KG):
                @pl.loop(0, CH, step=2)
                def _(i):
                    for u in range(2):
                        wspl = plsc.load_gather(
                            wv, [jnp.full((16,), slot, jnp.int32),
                                 jnp.full((16,), c, jnp.int32),
                                 jnp.full((16,), i + u, jnp.int32)])
                        for j in range(NJ):
                            sl = pl.ds(j * 16, 16)
                            rows[slot, c, i + u, sl] = \
                                rows[slot, c, i + u, sl] * wspl

                pltpu.async_copy(rows.at[slot, c],
                                 aggsh.at[sidx.at[slot, c]],
                                 ssems[slot], add=True)

        fire(0, 0, gsem0)

        @pl.loop(0, G // 2)
        def _(gg):
            for slot in (0, 1):
                g = gg * 2 + slot
                drain(slot, gsems[slot])

                @pl.when(g >= 1)
                def _():
                    drain_scatter(1 - slot)

                @pl.when(g + 1 < G)
                def _():
                    fire(g + 1, 1 - slot, gsems[1 - slot])

                scale_and_scatter(slot)

        if G % 2 == 1:
            drain(0, gsem0)
            if G > 1:
                drain_scatter(1)
            scale_and_scatter(0)

        drain_scatter((G - 1) % 2)
        plsc.subcore_barrier()

        pltpu.sync_copy(
            aggsh.at[pl.ds(sid * rows_per_tile, rows_per_tile)],
            out_hbm.at[cid, pl.ds(sid * rows_per_tile, rows_per_tile)])

    return sck(src3, dst3, w3, x2)


def _node_body(x_ref, apa_ref, apb_ref, deg_ref, w1x_ref, w1al_ref, w1ar_ref,
               w1d_ref, bm1_ref, w2_ref, bm2_ref, g_ref, b_ref, o_ref):
    xb = x_ref[...]
    deg = deg_ref[...]
    inv = 1.0 / jnp.maximum(deg, 1.0)
    aggl = (apa_ref[0] + apb_ref[0]) * inv
    aggr = (apa_ref[1] + apb_ref[1]) * inv
    h = (jnp.dot(xb, w1x_ref[...], preferred_element_type=jnp.float32)
         + jnp.dot(aggl, w1al_ref[...], preferred_element_type=jnp.float32)
         + jnp.dot(aggr, w1ar_ref[...], preferred_element_type=jnp.float32)
         + deg * w1d_ref[...] + bm1_ref[...])
    h = jnp.maximum(h, 0.0)
    out = jnp.dot(h, w2_ref[...], preferred_element_type=jnp.float32) + bm2_ref[...]
    y = xb + out
    mean = jnp.mean(y, axis=1, keepdims=True)
    yc = y - mean
    var = jnp.mean(yc * yc, axis=1, keepdims=True)
    o_ref[...] = yc * lax.rsqrt(var + 1e-5) * g_ref[...] + b_ref[...]


def _node_mlp(x, aggpa, aggpb, deg2, w1x, w1al, w1ar, w1d, bm1, w2, bm2,
              gamma, beta):
    N, D = x.shape
    DH = aggpa.shape[2]
    H = w1x.shape[1]
    BN = 2000
    grid = N // BN
    full = lambda shape: pl.BlockSpec(shape, lambda i: (0, 0))
    return pl.pallas_call(
        _node_body,
        grid=(grid,),
        in_specs=[
            pl.BlockSpec((BN, D), lambda i: (i, 0)),
            pl.BlockSpec((2, BN, DH), lambda i: (0, i, 0)),
            pl.BlockSpec((2, BN, DH), lambda i: (0, i, 0)),
            pl.BlockSpec((BN, 1), lambda i: (i, 0)),
            full((D, H)), full((DH, H)), full((DH, H)), full((1, H)),
            full((1, H)), full((H, D)), full((1, D)), full((1, D)), full((1, D)),
        ],
        out_specs=pl.BlockSpec((BN, D), lambda i: (i, 0)),
        out_shape=jax.ShapeDtypeStruct((N, D), jnp.float32),
    )(x, aggpa, aggpb, deg2, w1x, w1al, w1ar, w1d, bm1, w2, bm2, gamma, beta)


def kernel(x, edge_index, edge_feat, degrees,
           W1, b1, W2, b2, Wm1, bm1, Wm2, bm2, gamma, beta):
    N, D = x.shape
    E = edge_index.shape[1]
    H = W1.shape[1]
    NS = 16
    E2 = E // 2
    per_t = E2 // NS
    CH = 80
    n_chunks = per_t // CH
    DH = D // 2

    x2 = x.reshape(N, 2, DH).transpose(1, 0, 2)
    eft = edge_feat.T
    w1t, b1c = W1.T, b1.reshape(H, 1)
    b2c = b2.reshape(1, 1)

    aggps = []
    for ph in range(2):
        efp = lax.slice_in_dim(eft, ph * E2, (ph + 1) * E2, axis=1)
        w = _edge_weights(efp, w1t, b1c, W2, b2c)
        src3 = lax.slice_in_dim(edge_index[0], ph * E2,
                                (ph + 1) * E2).reshape(NS, n_chunks, CH)
        dst3 = lax.slice_in_dim(edge_index[1], ph * E2,
                                (ph + 1) * E2).reshape(NS, n_chunks, CH)
        w3 = w.reshape(NS, n_chunks, CH)
        aggps.append(_sc_message_pass(src3, dst3, w3, x2))

    y = _node_mlp(x, aggps[0], aggps[1], degrees.reshape(N, 1),
                  Wm1[:D], Wm1[D:D + DH], Wm1[D + DH:2 * D], Wm1[2 * D:],
                  bm1.reshape(1, H), Wm2, bm2.reshape(1, D),
                  gamma.reshape(1, D), beta.reshape(1, D))
    return y
